# baseline (speedup 1.0000x reference)
import dataclasses

import jax
import jax.numpy as jnp
import numpy as np
from jax.experimental import pallas as pl
from jax.experimental.pallas import tpu as pltpu
from jax.experimental.pallas import tpu_sc as plsc

NF = 26
VOCAB = 100000
EMB = 64
BATCH = 4096
NC = NF + 1

_NUNITS = NF * EMB
_NWORK = 32
_UPW = _NUNITS // _NWORK


def _sc_gather_planes(t3, idxT):
    vec_mesh = plsc.VectorSubcoreMesh(core_axis_name="c", subcore_axis_name="s")
    cp = pltpu.CompilerParams(use_tc_tiling_on_sc=True)
    if "needs_layout_passes" in pltpu.CompilerParams.__dataclass_fields__:
        cp = dataclasses.replace(cp, needs_layout_passes=False)

    @pl.kernel(
        out_type=jax.ShapeDtypeStruct((NF, EMB, BATCH), jnp.float32),
        mesh=vec_mesh,
        compiler_params=cp,
        scratch_types=[
            pltpu.VMEM((VOCAB,), jnp.float32),
            pltpu.VMEM((BATCH,), jnp.int32),
            pltpu.VMEM((BATCH,), jnp.float32),
        ],
    )
    def kern(t_hbm, i_hbm, o_hbm, plane_v, idx_v, out_v):
        wid = jax.lax.axis_index("c") * 16 + jax.lax.axis_index("s")

        @pl.loop(0, _UPW)
        def _unit(k):
            u = wid * _UPW + k
            f = u // EMB
            j = u % EMB
            pltpu.sync_copy(i_hbm.at[f], idx_v)
            pltpu.sync_copy(t_hbm.at[f, j], plane_v)

            @pl.loop(0, BATCH // 16)
            def _grp(g):
                iv = idx_v[pl.ds(g * 16, 16)]
                out_v[pl.ds(g * 16, 16)] = plsc.load_gather(plane_v, [iv])

            pltpu.sync_copy(out_v, o_hbm.at[f, j])

    return kern(t3, idxT)


_RB = 512


def _dot1(a, b):
    return jnp.dot(a.astype(jnp.bfloat16), b.astype(jnp.bfloat16),
                   preferred_element_type=jnp.float32)


def _gram1(c):
    ch = c.astype(jnp.bfloat16)
    return jax.lax.dot_general(ch, ch, (((2,), (2,)), ((0,), (0,))),
                               preferred_element_type=jnp.float32)


def _tc_body(d_ref, g_ref, w0, b0, w1, b1, w2, b2, wa, w0k, w2b, fb0,
             fw1, fb1, fw2, fb2, o_ref):
    h = jnp.maximum(_dot1(d_ref[...], w0[...]) + b0[...], 0.0)
    h = jnp.maximum(_dot1(h, w1[...]) + b1[...], 0.0)
    ed = jnp.maximum(_dot1(h, w2[...]) + b2[...], 0.0)
    gb = g_ref[...].astype(jnp.bfloat16)
    edb = ed.astype(jnp.bfloat16)
    d0 = jax.lax.dot_general(edb, gb, (((1,), (2,)), ((0,), (0,))),
                             preferred_element_type=jnp.float32)
    gram = jax.lax.dot_general(gb, gb, (((2,), (2,)), ((0,), (0,))),
                               preferred_element_type=jnp.float32)
    y = (_dot1(ed, wa[...]) + _dot1(d0, w0k[...])
         + _dot1(gram.reshape(_RB, NF * NF), w2b[...]) + fb0[...])
    g = jnp.maximum(y, 0.0)
    g = jnp.maximum(_dot1(g, fw1[...]) + fb1[...], 0.0)
    o_ref[...] = _dot1(g, fw2[...]) + fb2[...]


_iu = np.triu_indices(NC, k=1)
_pm = np.zeros((NC, NC), np.int32)
_pm[_iu] = np.arange(NC * (NC - 1) // 2, dtype=np.int32)
_pm = _pm + _pm.T
_PM0 = _pm[0, 1:]
_PM11 = _pm[1:, 1:].reshape(-1)
_OFFDIAG = (1.0 - np.eye(NF, dtype=np.float32)).reshape(-1, 1)


def _tc_forward(dense, g3, w0, b0, w1, b1, w2, b2, fw0, fb0, fw1, fb1, fw2, fb2):
    wa = fw0[:EMB]
    w0k = jnp.take(fw0[EMB:], _PM0, axis=0)
    w2b = 0.5 * jnp.take(fw0[EMB:], _PM11, axis=0) * _OFFDIAG
    row = lambda v: v.reshape(1, -1)
    grid = BATCH // _RB
    full = lambda a: pl.BlockSpec(a.shape, lambda i: (0,) * a.ndim)
    args = (dense, g3, w0, row(b0), w1, row(b1), w2, row(b2), wa, w0k, w2b,
            row(fb0), fw1, row(fb1), fw2, row(fb2))
    in_specs = [
        pl.BlockSpec((_RB, dense.shape[1]), lambda i: (i, 0)),
        pl.BlockSpec((_RB, NF, EMB), lambda i: (i, 0, 0)),
    ] + [full(a) for a in args[2:]]
    return pl.pallas_call(
        _tc_body,
        grid=(grid,),
        in_specs=in_specs,
        out_specs=pl.BlockSpec((_RB, 1), lambda i: (i, 0)),
        out_shape=jax.ShapeDtypeStruct((BATCH, 1), jnp.float32),
    )(*args)


def kernel(dense_features, sparse_features, emb_tables,
           w0, b0, w1, b1, w2, b2,
           fw0, fb0, fw1, fb1, fw2, fb2):
    t3 = jnp.transpose(emb_tables, (0, 2, 1))
    idxT = sparse_features.astype(jnp.int32).T
    p = _sc_gather_planes(t3, idxT)
    g3 = jnp.transpose(p, (2, 0, 1))
    return _tc_forward(dense_features, g3, w0, b0, w1, b1, w2, b2,
                       fw0, fb0, fw1, fb1, fw2, fb2)


# speedup vs baseline: 1.0185x; 1.0185x over previous
import dataclasses

import jax
import jax.numpy as jnp
import numpy as np
from jax.experimental import pallas as pl
from jax.experimental.pallas import tpu as pltpu
from jax.experimental.pallas import tpu_sc as plsc

NF = 26
VOCAB = 100000
EMB = 64
BATCH = 4096
NC = NF + 1

_NUNITS = NF * EMB
_NWORK = 32
_UPW = _NUNITS // _NWORK


def _sc_gather_planes(t3, idxT):
    vec_mesh = plsc.VectorSubcoreMesh(core_axis_name="c", subcore_axis_name="s")
    cp = pltpu.CompilerParams(use_tc_tiling_on_sc=True)
    if "needs_layout_passes" in pltpu.CompilerParams.__dataclass_fields__:
        cp = dataclasses.replace(cp, needs_layout_passes=False)

    @pl.kernel(
        out_type=jax.ShapeDtypeStruct((NF, EMB, BATCH), jnp.float32),
        mesh=vec_mesh,
        compiler_params=cp,
        scratch_types=[
            pltpu.VMEM((VOCAB,), jnp.float32),
            pltpu.VMEM((BATCH,), jnp.int32),
            pltpu.VMEM((BATCH,), jnp.float32),
        ],
    )
    def kern(t_hbm, i_hbm, o_hbm, plane_v, idx_v, out_v):
        wid = jax.lax.axis_index("c") * 16 + jax.lax.axis_index("s")

        @pl.loop(0, _UPW)
        def _unit(k):
            u = wid * _UPW + k
            f = u // EMB
            j = u % EMB
            pltpu.sync_copy(i_hbm.at[f], idx_v)
            pltpu.sync_copy(t_hbm.at[f, j], plane_v)

            @pl.loop(0, BATCH // 16)
            def _grp(g):
                iv = idx_v[pl.ds(g * 16, 16)]
                out_v[pl.ds(g * 16, 16)] = plsc.load_gather(plane_v, [iv])

            pltpu.sync_copy(out_v, o_hbm.at[f, j])

    return kern(t3, idxT)


_RB = 512


def _dot1(a, b):
    return jnp.dot(a.astype(jnp.bfloat16), b.astype(jnp.bfloat16),
                   preferred_element_type=jnp.float32)


def _gram1(c):
    ch = c.astype(jnp.bfloat16)
    return jax.lax.dot_general(ch, ch, (((2,), (2,)), ((0,), (0,))),
                               preferred_element_type=jnp.float32)


def _tc_body(d_ref, g_ref, w0, b0, w1, b1, w2, b2, wcat, fb0, fw1, fb1,
             fw2, fb2, o_ref):
    h = jnp.maximum(_dot1(d_ref[...], w0[...]) + b0[...], 0.0)
    h = jnp.maximum(_dot1(h, w1[...]) + b1[...], 0.0)
    ed = jnp.maximum(_dot1(h, w2[...]) + b2[...], 0.0)
    c = jnp.concatenate([ed[:, None, :], g_ref[...]], axis=1)
    gram = _gram1(c)
    cat = jnp.concatenate([ed, gram.reshape(_RB, NC * NC)], axis=1)
    g = jnp.maximum(_dot1(cat, wcat[...]) + fb0[...], 0.0)
    g = jnp.maximum(_dot1(g, fw1[...]) + fb1[...], 0.0)
    o_ref[...] = _dot1(g, fw2[...]) + fb2[...]


_iu = np.triu_indices(NC, k=1)
_pm = np.zeros((NC, NC), np.int32)
_pm[_iu] = np.arange(NC * (NC - 1) // 2, dtype=np.int32)
_pm = _pm + _pm.T
_PAIR_MAP = _pm.reshape(-1)
_OFFDIAG = (1.0 - np.eye(NC, dtype=np.float32)).reshape(-1, 1)


def _tc_forward(dense, g3, w0, b0, w1, b1, w2, b2, fw0, fb0, fw1, fb1, fw2, fb2):
    w2i = 0.5 * jnp.take(fw0[EMB:], _PAIR_MAP, axis=0) * _OFFDIAG
    wcat = jnp.concatenate([fw0[:EMB], w2i], axis=0)
    row = lambda v: v.reshape(1, -1)
    grid = BATCH // _RB
    full = lambda a: pl.BlockSpec(a.shape, lambda i: (0,) * a.ndim)
    args = (dense, g3, w0, row(b0), w1, row(b1), w2, row(b2), wcat, row(fb0),
            fw1, row(fb1), fw2, row(fb2))
    in_specs = [
        pl.BlockSpec((_RB, dense.shape[1]), lambda i: (i, 0)),
        pl.BlockSpec((_RB, NF, EMB), lambda i: (i, 0, 0)),
    ] + [full(a) for a in args[2:]]
    return pl.pallas_call(
        _tc_body,
        grid=(grid,),
        in_specs=in_specs,
        out_specs=pl.BlockSpec((_RB, 1), lambda i: (i, 0)),
        out_shape=jax.ShapeDtypeStruct((BATCH, 1), jnp.float32),
    )(*args)


def kernel(dense_features, sparse_features, emb_tables,
           w0, b0, w1, b1, w2, b2,
           fw0, fb0, fw1, fb1, fw2, fb2):
    t3 = jnp.transpose(emb_tables, (0, 2, 1))
    idxT = sparse_features.astype(jnp.int32).T
    p = _sc_gather_planes(t3, idxT)
    g3 = jnp.transpose(p, (2, 0, 1))
    return _tc_forward(dense_features, g3, w0, b0, w1, b1, w2, b2,
                       fw0, fb0, fw1, fb1, fw2, fb2)
